# baseline (speedup 1.0000x reference)
import functools

import jax
import jax.numpy as jnp
from jax import lax
from jax.experimental import pallas as pl
from jax.experimental.pallas import tpu as pltpu
from jax.experimental.pallas import tpu_sc as plsc

_LANES = 16
_CHUNK = 4000


@functools.lru_cache(maxsize=None)
def _build(n_nodes, n_edges):
    info = plsc.get_sparse_core_info()
    nc, ns = info.num_cores, info.num_subcores
    nw = nc * ns
    epw = n_edges // nw
    assert n_edges % nw == 0
    assert epw % (2 * _CHUNK) == 0
    nch = epw // _CHUNK
    nvec = _CHUNK // _LANES
    mesh = plsc.VectorSubcoreMesh(core_axis_name="c", subcore_axis_name="s")

    @functools.partial(
        pl.kernel,
        out_type=(
            jax.ShapeDtypeStruct((n_edges,), jnp.int32),
            jax.ShapeDtypeStruct((n_edges,), jnp.int32),
        ),
        mesh=mesh,
        compiler_params=pltpu.CompilerParams(needs_layout_passes=False),
        scratch_types=[
            pltpu.VMEM((n_nodes,), jnp.int32),
            pltpu.VMEM((_CHUNK,), jnp.int32),
            pltpu.VMEM((_CHUNK,), jnp.int32),
            pltpu.VMEM((_CHUNK,), jnp.int32),
            pltpu.VMEM((_CHUNK,), jnp.int32),
            pltpu.VMEM((_CHUNK,), jnp.int32),
            pltpu.VMEM((_CHUNK,), jnp.int32),
            pltpu.SemaphoreType.DMA,
            pltpu.SemaphoreType.DMA,
            pltpu.SemaphoreType.DMA,
            pltpu.SemaphoreType.DMA,
            pltpu.SemaphoreType.DMA,
            pltpu.SemaphoreType.DMA,
        ],
    )
    def kern(scores_hbm, oid_hbm, ei_hbm, sout_hbm, eqout_hbm,
             table_v, idxa0_v, idxa1_v, idxb0_v, idxb1_v, out0_v, out1_v,
             sa0, sa1, sb0, sb1, so0, so1):
        wid = lax.axis_index("s") * nc + lax.axis_index("c")
        base = wid * epw
        idxa = (idxa0_v, idxa1_v)
        idxb = (idxb0_v, idxb1_v)
        out = (out0_v, out1_v)
        sa = (sa0, sa1)
        sb = (sb0, sb1)
        so = (so0, so1)

        def ring(fire_in, wait_in, wait_out, step):
            fire_in(0, 0)

            def body(i, carry):
                for b in range(2):
                    g = 2 * i + b
                    if b == 0:
                        fire_in(g + 1, 1)
                    else:
                        @pl.when(i < nch // 2 - 1)
                        def _():
                            fire_in(g + 1, 0)
                    wait_in(g, b)

                    @pl.when(i >= 1)
                    def _():
                        wait_out(g - 2, b)

                    step(g, b)
                return carry

            lax.fori_loop(0, nch // 2, body, 0)
            wait_out(nch - 2, 0)
            wait_out(nch - 1, 1)

        pltpu.sync_copy(scores_hbm, table_v)

        def fire_in1(g, slot):
            cb = base + g * _CHUNK
            pltpu.async_copy(ei_hbm.at[pl.ds(cb, _CHUNK)], idxa[slot],
                             sa[slot])

        def wait_in1(g, slot):
            cb = base + g * _CHUNK
            pltpu.make_async_copy(ei_hbm.at[pl.ds(cb, _CHUNK)], idxa[slot],
                                  sa[slot]).wait()

        def wait_out1(g, slot):
            cb = base + g * _CHUNK
            pltpu.make_async_copy(out[slot], sout_hbm.at[pl.ds(cb, _CHUNK)],
                                  so[slot]).wait()

        def step1(g, b):
            @plsc.parallel_loop(0, nvec, unroll=8)
            def _(j):
                idx = idxa[b][pl.ds(j * _LANES, _LANES)]
                out[b][pl.ds(j * _LANES, _LANES)] = plsc.load_gather(
                    table_v, [idx])
            cb = base + g * _CHUNK
            pltpu.async_copy(out[b], sout_hbm.at[pl.ds(cb, _CHUNK)], so[b])

        ring(fire_in1, wait_in1, wait_out1, step1)

        pltpu.sync_copy(oid_hbm, table_v)

        def fire_in2(g, slot):
            cb = base + g * _CHUNK
            pltpu.async_copy(ei_hbm.at[pl.ds(cb, _CHUNK)], idxa[slot],
                             sa[slot])
            pltpu.async_copy(ei_hbm.at[pl.ds(n_edges + cb, _CHUNK)],
                             idxb[slot], sb[slot])

        def wait_in2(g, slot):
            cb = base + g * _CHUNK
            pltpu.make_async_copy(ei_hbm.at[pl.ds(cb, _CHUNK)], idxa[slot],
                                  sa[slot]).wait()
            pltpu.make_async_copy(ei_hbm.at[pl.ds(n_edges + cb, _CHUNK)],
                                  idxb[slot], sb[slot]).wait()

        def wait_out2(g, slot):
            cb = base + g * _CHUNK
            pltpu.make_async_copy(out[slot], eqout_hbm.at[pl.ds(cb, _CHUNK)],
                                  so[slot]).wait()

        def step2(g, b):
            @plsc.parallel_loop(0, nvec, unroll=8)
            def _(j):
                s = idxa[b][pl.ds(j * _LANES, _LANES)]
                t = idxb[b][pl.ds(j * _LANES, _LANES)]
                a = plsc.load_gather(table_v, [s])
                c = plsc.load_gather(table_v, [t])
                out[b][pl.ds(j * _LANES, _LANES)] = (a == c).astype(jnp.int32)
            cb = base + g * _CHUNK
            pltpu.async_copy(out[b], eqout_hbm.at[pl.ds(cb, _CHUNK)], so[b])

        ring(fire_in2, wait_in2, wait_out2, step2)

    return kern


def kernel(scores, object_id, edge_index):
    n_nodes = scores.shape[0]
    n_edges = edge_index.shape[1]
    scores_i = lax.bitcast_convert_type(scores.reshape(-1), jnp.int32)
    ei_flat = edge_index.reshape(-1)
    sout, eqout = _build(n_nodes, n_edges)(scores_i, object_id, ei_flat)
    return (lax.bitcast_convert_type(sout, jnp.float32),
            eqout.astype(jnp.bool_))


# speedup vs baseline: 1.1812x; 1.1812x over previous
import functools

import jax
import jax.numpy as jnp
from jax import lax
from jax.experimental import pallas as pl
from jax.experimental.pallas import tpu as pltpu
from jax.experimental.pallas import tpu_sc as plsc

_LANES = 16
_CHUNK = 4000
_WIN = _CHUNK + 224


@functools.lru_cache(maxsize=None)
def _build(n_nodes, n_edges):
    info = plsc.get_sparse_core_info()
    nc, ns = info.num_cores, info.num_subcores
    nw = nc * ns
    epw = n_edges // nw
    assert n_edges % nw == 0
    assert epw % (2 * _CHUNK) == 0
    nch = epw // _CHUNK
    nvec = _CHUNK // _LANES
    mesh = plsc.VectorSubcoreMesh(core_axis_name="c", subcore_axis_name="s")

    @functools.partial(
        pl.kernel,
        out_type=(
            jax.ShapeDtypeStruct((n_edges,), jnp.int32),
            jax.ShapeDtypeStruct((n_edges,), jnp.int32),
        ),
        mesh=mesh,
        compiler_params=pltpu.CompilerParams(needs_layout_passes=False),
        scratch_types=[
            pltpu.VMEM((n_nodes,), jnp.int32),
            pltpu.VMEM((2, _WIN), jnp.int32),
            pltpu.VMEM((2, _WIN), jnp.int32),
            pltpu.VMEM((_CHUNK,), jnp.int32),
            pltpu.VMEM((_CHUNK,), jnp.int32),
            pltpu.SemaphoreType.DMA,
            pltpu.SemaphoreType.DMA,
            pltpu.SemaphoreType.DMA,
            pltpu.SemaphoreType.DMA,
        ],
    )
    def kern(scores_hbm, oid_hbm, ei_hbm, sout_hbm, eqout_hbm,
             table_v, idx0_v, idx1_v, out0_v, out1_v,
             sa0, sa1, so0, so1):
        wid = lax.axis_index("s") * nc + lax.axis_index("c")
        base = wid * epw
        idx = (idx0_v, idx1_v)
        out = (out0_v, out1_v)
        sa = (sa0, sa1)
        so = (so0, so1)

        def win_start(g):
            cb = base + g * _CHUNK
            s = jnp.minimum(cb - lax.rem(cb, 128), n_edges - _WIN)
            return pl.multiple_of(s, 128)

        def fire_in(g, slot):
            pltpu.async_copy(ei_hbm.at[:, pl.ds(win_start(g), _WIN)],
                             idx[slot], sa[slot])

        def wait_in(g, slot):
            pltpu.make_async_copy(ei_hbm.at[:, pl.ds(win_start(g), _WIN)],
                                  idx[slot], sa[slot]).wait()

        def ring(out_hbm, step):
            def wait_out(g, slot):
                cb = base + g * _CHUNK
                pltpu.make_async_copy(out[slot],
                                      out_hbm.at[pl.ds(cb, _CHUNK)],
                                      so[slot]).wait()

            fire_in(0, 0)

            def body(i, carry):
                for b in range(2):
                    g = 2 * i + b
                    if b == 0:
                        fire_in(g + 1, 1)
                    else:
                        @pl.when(i < nch // 2 - 1)
                        def _():
                            fire_in(g + 1, 0)
                    wait_in(g, b)

                    @pl.when(i >= 1)
                    def _():
                        wait_out(g - 2, b)

                    step(g, b)
                    cb = base + g * _CHUNK
                    pltpu.async_copy(out[b], out_hbm.at[pl.ds(cb, _CHUNK)],
                                     so[b])
                return carry

            lax.fori_loop(0, nch // 2, body, 0)
            wait_out(nch - 2, 0)
            wait_out(nch - 1, 1)

        pltpu.sync_copy(scores_hbm, table_v)

        def step1(g, b):
            off = base + g * _CHUNK - win_start(g)

            @plsc.parallel_loop(0, nvec, unroll=8)
            def _(j):
                s = idx[b][0, pl.ds(off + j * _LANES, _LANES)]
                out[b][pl.ds(j * _LANES, _LANES)] = plsc.load_gather(
                    table_v, [s])

        ring(sout_hbm, step1)

        pltpu.sync_copy(oid_hbm, table_v)

        def step2(g, b):
            off = base + g * _CHUNK - win_start(g)

            @plsc.parallel_loop(0, nvec, unroll=8)
            def _(j):
                s = idx[b][0, pl.ds(off + j * _LANES, _LANES)]
                t = idx[b][1, pl.ds(off + j * _LANES, _LANES)]
                a = plsc.load_gather(table_v, [s])
                c = plsc.load_gather(table_v, [t])
                out[b][pl.ds(j * _LANES, _LANES)] = (a == c).astype(jnp.int32)

        ring(eqout_hbm, step2)

    return kern


def kernel(scores, object_id, edge_index):
    n_nodes = scores.shape[0]
    n_edges = edge_index.shape[1]
    scores_i = lax.bitcast_convert_type(scores.reshape(-1), jnp.int32)
    sout, eqout = _build(n_nodes, n_edges)(scores_i, object_id, edge_index)
    return (lax.bitcast_convert_type(sout, jnp.float32),
            eqout.astype(jnp.bool_))
